# Initial kernel scaffold; baseline (speedup 1.0000x reference)
import jax
import jax.numpy as jnp
from jax.experimental import pallas as pl


def kernel(dense_x, sparse_x, emb_tables, W1, b1, W2, b2, W3, b3, W4, b4):
    raise NotImplementedError("write your pallas kernel here")


# speedup vs baseline: 1.8236x; 1.8236x over previous
import functools

import jax
import jax.numpy as jnp
from jax import lax
from jax.experimental import pallas as pl
from jax.experimental.pallas import tpu as pltpu
from jax.experimental.pallas import tpu_sc as plsc

NUM_FIELDS = 26
VOCAB = 100000
EMBED_DIM = 32
DENSE_DIM = 13
BATCH = 16384
CONCAT = NUM_FIELDS * EMBED_DIM

NC, NS = 2, 16
NW = NC * NS
TOTAL = BATCH * NUM_FIELDS
PER_W = TOTAL // NW
IDX_MINOR = 128
CHUNK = 512
GPC = CHUNK // IDX_MINOR
N_CHUNKS = PER_W // CHUNK


def _sc_gather(sidx, tbl):
    mesh = plsc.VectorSubcoreMesh(core_axis_name="c", subcore_axis_name="s")

    @functools.partial(
        pl.kernel,
        mesh=mesh,
        out_type=jax.ShapeDtypeStruct((TOTAL, EMBED_DIM), jnp.float32),
        compiler_params=pltpu.CompilerParams(use_tc_tiling_on_sc=False),
        scratch_types=[
            pltpu.VMEM((GPC, IDX_MINOR), jnp.int32),
            pltpu.VMEM((CHUNK, EMBED_DIM), jnp.float32),
            pltpu.SemaphoreType.DMA,
        ],
    )
    def gather_kernel(sidx_hbm, tbl_hbm, out_hbm, idx_v, rows_v, sem):
        wid = lax.axis_index("s") * NC + lax.axis_index("c")

        def body(c, carry):
            pltpu.sync_copy(sidx_hbm.at[wid, c], idx_v)
            cps = [
                pltpu.async_copy(
                    tbl_hbm.at[idx_v.at[j]],
                    rows_v.at[pl.ds(j * IDX_MINOR, IDX_MINOR)],
                    sem,
                )
                for j in range(GPC)
            ]
            for cp in cps:
                cp.wait()
            pltpu.sync_copy(
                rows_v, out_hbm.at[pl.ds((wid * N_CHUNKS + c) * CHUNK, CHUNK)]
            )
            return carry

        lax.fori_loop(0, N_CHUNKS, body, 0)

    return gather_kernel(sidx, tbl)


def _dense_body(dx_ref, g_ref, w1_ref, b1_ref, w2_ref, b2_ref, w3a_ref,
                w3b_ref, b3_ref, w4_ref, b4_ref, out_ref):
    h = jnp.maximum(
        jnp.dot(dx_ref[...], w1_ref[...], preferred_element_type=jnp.float32)
        + b1_ref[...], 0.0)
    d = jnp.dot(h, w2_ref[...], preferred_element_type=jnp.float32) + b2_ref[...]
    t = (jnp.dot(g_ref[...], w3a_ref[...], preferred_element_type=jnp.float32)
         + jnp.dot(d, w3b_ref[...], preferred_element_type=jnp.float32)
         + b3_ref[...])
    h2 = jnp.maximum(t, 0.0)
    z = jnp.dot(h2, w4_ref[...], preferred_element_type=jnp.float32) + b4_ref[...]
    out_ref[...] = 1.0 / (1.0 + jnp.exp(-z))


_BT = 2048


def _dense_forward(dense_x, g, W1, b1, W2, b2, W3a, W3b, b3, W4, b4):
    fixed = lambda t: (0, 0)
    tiled = lambda t: (t, 0)
    return pl.pallas_call(
        _dense_body,
        grid=(BATCH // _BT,),
        in_specs=[
            pl.BlockSpec((_BT, DENSE_DIM), tiled),
            pl.BlockSpec((_BT, CONCAT), tiled),
            pl.BlockSpec((DENSE_DIM, 8), fixed),
            pl.BlockSpec((1, 8), fixed),
            pl.BlockSpec((8, EMBED_DIM), fixed),
            pl.BlockSpec((1, EMBED_DIM), fixed),
            pl.BlockSpec((CONCAT, 16), fixed),
            pl.BlockSpec((EMBED_DIM, 16), fixed),
            pl.BlockSpec((1, 16), fixed),
            pl.BlockSpec((16, 1), fixed),
            pl.BlockSpec((1, 1), fixed),
        ],
        out_specs=pl.BlockSpec((_BT, 1), tiled),
        out_shape=jax.ShapeDtypeStruct((BATCH, 1), jnp.float32),
    )(dense_x, g, W1, b1, W2, b2, W3a, W3b, b3, W4, b4)


def kernel(dense_x, sparse_x, emb_tables, W1, b1, W2, b2, W3, b3, W4, b4):
    offs = (jnp.arange(NUM_FIELDS, dtype=jnp.int32) * VOCAB)[None, :]
    sidx = (sparse_x + offs).reshape(NW, N_CHUNKS, GPC, IDX_MINOR)
    tbl = emb_tables.reshape(NUM_FIELDS * VOCAB, EMBED_DIM)
    g = _sc_gather(sidx, tbl).reshape(BATCH, CONCAT)
    return _dense_forward(
        dense_x, g, W1, b1.reshape(1, 8), W2, b2.reshape(1, EMBED_DIM),
        W3[:CONCAT], W3[CONCAT:], b3.reshape(1, 16), W4, b4.reshape(1, 1))
